# baseline (speedup 1.0000x reference)
import functools

import numpy as np
import jax
import jax.numpy as jnp
from jax import lax
from jax.experimental import pallas as pl
from jax.experimental.pallas import tpu as pltpu
from jax.experimental.pallas import tpu_sc as plsc

_NC = 2
_NS = 16
_NW = _NC * _NS

_GL = 128
_KG = 10
_BLK = _GL * _KG


def _make_gather(vocab: int, batch: int, dim: int):
    assert batch % (_NW * 2 * _BLK) == 0
    rows_per_w = batch // _NW
    irows_per_w = rows_per_w // _GL
    npair = rows_per_w // (2 * _BLK)

    mesh = plsc.VectorSubcoreMesh(core_axis_name="c", subcore_axis_name="s")

    @functools.partial(
        pl.kernel,
        out_type=jax.ShapeDtypeStruct((batch, dim), jnp.float32),
        mesh=mesh,
        scratch_types=[
            pltpu.VMEM((irows_per_w, _GL), jnp.int32),
            pltpu.VMEM((_BLK, dim), jnp.float32),
            pltpu.VMEM((_BLK, dim), jnp.float32),
            pltpu.SemaphoreType.DMA,
            pltpu.SemaphoreType.DMA,
            pltpu.SemaphoreType.DMA,
            pltpu.SemaphoreType.DMA,
        ],
        compiler_params=pltpu.CompilerParams(use_tc_tiling_on_sc=False),
    )
    def gather_kernel(table_hbm, idx_hbm, out_hbm, idx_v, buf0, buf1,
                      gsem0, gsem1, wsem0, wsem1):
        wid = lax.axis_index("s") * _NC + lax.axis_index("c")
        row0 = wid * rows_per_w
        pltpu.sync_copy(idx_hbm.at[pl.ds(wid * irows_per_w, irows_per_w)], idx_v)

        def fire_gather(blk, buf, sem):
            for j in range(_KG):
                pltpu.async_copy(
                    table_hbm.at[idx_v.at[blk * _KG + j]],
                    buf.at[pl.ds(j * _GL, _GL)],
                    sem,
                )

        def drain_gather(buf, sem):
            pltpu.make_async_copy(table_hbm.at[pl.ds(0, _BLK)], buf, sem).wait()

        def fire_write(blk, buf, sem):
            pltpu.async_copy(buf, out_hbm.at[pl.ds(row0 + blk * _BLK, _BLK)], sem)

        def drain_write(buf, sem):
            pltpu.make_async_copy(buf, out_hbm.at[pl.ds(0, _BLK)], sem).wait()

        fire_gather(0, buf0, gsem0)

        def step(p, carry):
            @pl.when(p >= 1)
            def _():
                drain_write(buf1, wsem1)
            fire_gather(2 * p + 1, buf1, gsem1)
            drain_gather(buf0, gsem0)
            fire_write(2 * p, buf0, wsem0)

            @pl.when(p <= npair - 2)
            def _():
                drain_write(buf0, wsem0)
                fire_gather(2 * p + 2, buf0, gsem0)
            drain_gather(buf1, gsem1)
            fire_write(2 * p + 1, buf1, wsem1)
            return carry

        lax.fori_loop(0, npair, step, None)
        drain_write(buf0, wsem0)
        drain_write(buf1, wsem1)

    return gather_kernel


_CB = 16
_CR = _CB * 50
_CP = _CR // 4
_HB = _CB // 2
_HR = _HB * 50
_HP = 56


def _make_format(b_sz: int, h_sz: int, dim: int):
    assert h_sz == 50 and dim == 32
    b_per_w = b_sz // _NW
    nchunk = b_per_w // _CB

    mesh = plsc.VectorSubcoreMesh(core_axis_name="c", subcore_axis_name="s")

    @functools.partial(
        pl.kernel,
        out_type=jax.ShapeDtypeStruct((b_sz, h_sz, dim), jnp.float32),
        mesh=mesh,
        scratch_types=[
            pltpu.VMEM((_CP + 8, 128), jnp.float32),
            pltpu.VMEM((_HB * _HP, dim), jnp.float32),
        ],
        compiler_params=pltpu.CompilerParams(
            use_tc_tiling_on_sc=True, needs_layout_passes=False
        ),
    )
    def format_kernel(g_hbm, out_hbm, rbuf, cbuf):
        wid = lax.axis_index("s") * _NC + lax.axis_index("c")
        b0 = wid * b_per_w
        p0 = b0 * h_sz // 4

        def step(ch, carry):
            off = pl.multiple_of(p0 + ch * _CP, 8)
            pltpu.sync_copy(g_hbm.at[pl.ds(off, _CP)], rbuf.at[pl.ds(0, _CP)])
            iota = lax.iota(jnp.int32, 16)
            zero16 = lax.shift_right_logical(iota, 4)
            for half in range(2):

                def gstep(g, carry2):
                    r = iota + 16 * g + half * _HR
                    q16 = lax.shift_right_logical(r, 2)
                    c16 = lax.shift_left(r & 3, 5)
                    rb = lax.shift_right_logical(r * 83887, 22)
                    s16 = r + 6 * rb - half * _HB * _HP
                    for c in range(dim):
                        v = plsc.load_gather(rbuf, [q16, c16 + c])
                        plsc.store_scatter(cbuf, [s16, zero16 + c], v)
                    return carry2

                lax.fori_loop(0, _HR // 16, gstep, None)
                for k in range(_HB):
                    pltpu.sync_copy(
                        cbuf.at[pl.ds(k * _HP, h_sz)],
                        out_hbm.at[b0 + ch * _CB + half * _HB + k],
                    )
            return carry

        lax.fori_loop(0, nchunk, step, None)

    return format_kernel


def kernel(input, emb_matrix):
    b_sz, h_sz = input.shape
    batch = b_sz * h_sz
    vocab, dim = emb_matrix.shape
    idx = input.reshape(batch // _GL, _GL).astype(jnp.int32)
    flat = _make_gather(vocab, batch, dim)(emb_matrix, idx)
    flat_b = jax.lax.optimization_barrier(flat)
    return flat_b.reshape(b_sz, h_sz, dim)


# speedup vs baseline: 1.6279x; 1.6279x over previous
import functools

import numpy as np
import jax
import jax.numpy as jnp
from jax import lax
from jax.experimental import pallas as pl
from jax.experimental.pallas import tpu as pltpu
from jax.experimental.pallas import tpu_sc as plsc

_NC = 2
_NS = 16
_NW = _NC * _NS

_GL = 128
_KG = 10
_BLK = _GL * _KG


def _make_gather(vocab: int, batch: int, dim: int):
    assert batch % (_NW * 2 * _BLK) == 0
    rows_per_w = batch // _NW
    irows_per_w = rows_per_w // _GL
    npair = rows_per_w // (2 * _BLK)

    mesh = plsc.VectorSubcoreMesh(core_axis_name="c", subcore_axis_name="s")

    @functools.partial(
        pl.kernel,
        out_type=jax.ShapeDtypeStruct((batch, dim), jnp.float32),
        mesh=mesh,
        scratch_types=[
            pltpu.VMEM((irows_per_w, _GL), jnp.int32),
            pltpu.VMEM((_BLK, dim), jnp.float32),
            pltpu.VMEM((_BLK, dim), jnp.float32),
            pltpu.SemaphoreType.DMA,
            pltpu.SemaphoreType.DMA,
            pltpu.SemaphoreType.DMA,
            pltpu.SemaphoreType.DMA,
        ],
        compiler_params=pltpu.CompilerParams(use_tc_tiling_on_sc=False),
    )
    def gather_kernel(table_hbm, idx_hbm, out_hbm, idx_v, buf0, buf1,
                      gsem0, gsem1, wsem0, wsem1):
        wid = lax.axis_index("s") * _NC + lax.axis_index("c")
        row0 = wid * rows_per_w
        pltpu.sync_copy(idx_hbm.at[pl.ds(wid * irows_per_w, irows_per_w)], idx_v)

        def fire_gather(blk, buf, sem):
            for j in range(_KG):
                pltpu.async_copy(
                    table_hbm.at[idx_v.at[blk * _KG + j]],
                    buf.at[pl.ds(j * _GL, _GL)],
                    sem,
                )

        def drain_gather(buf, sem):
            pltpu.make_async_copy(table_hbm.at[pl.ds(0, _BLK)], buf, sem).wait()

        def fire_write(blk, buf, sem):
            pltpu.async_copy(buf, out_hbm.at[pl.ds(row0 + blk * _BLK, _BLK)], sem)

        def drain_write(buf, sem):
            pltpu.make_async_copy(buf, out_hbm.at[pl.ds(0, _BLK)], sem).wait()

        fire_gather(0, buf0, gsem0)

        def step(p, carry):
            @pl.when(p >= 1)
            def _():
                drain_write(buf1, wsem1)
            fire_gather(2 * p + 1, buf1, gsem1)
            drain_gather(buf0, gsem0)
            fire_write(2 * p, buf0, wsem0)

            @pl.when(p <= npair - 2)
            def _():
                drain_write(buf0, wsem0)
                fire_gather(2 * p + 2, buf0, gsem0)
            drain_gather(buf1, gsem1)
            fire_write(2 * p + 1, buf1, wsem1)
            return carry

        lax.fori_loop(0, npair, step, None)
        drain_write(buf0, wsem0)
        drain_write(buf1, wsem1)

    return gather_kernel


_CB = 16
_CR = _CB * 50
_CP = _CR // 4
_HB = _CB // 2
_HR = _HB * 50
_HP = 56


def _make_format(b_sz: int, h_sz: int, dim: int):
    assert h_sz == 50 and dim == 32
    b_per_w = b_sz // _NW
    nchunk = b_per_w // _CB

    mesh = plsc.VectorSubcoreMesh(core_axis_name="c", subcore_axis_name="s")

    @functools.partial(
        pl.kernel,
        out_type=jax.ShapeDtypeStruct((b_sz, h_sz, dim), jnp.float32),
        mesh=mesh,
        scratch_types=[
            pltpu.VMEM((_CP + 8, 128), jnp.float32),
            pltpu.VMEM((_HB * _HP, dim), jnp.float32),
        ],
        compiler_params=pltpu.CompilerParams(
            use_tc_tiling_on_sc=True, needs_layout_passes=False
        ),
    )
    def format_kernel(g_hbm, out_hbm, rbuf, cbuf):
        wid = lax.axis_index("s") * _NC + lax.axis_index("c")
        b0 = wid * b_per_w
        p0 = b0 * h_sz // 4

        def step(ch, carry):
            off = pl.multiple_of(p0 + ch * _CP, 8)
            pltpu.sync_copy(g_hbm.at[pl.ds(off, _CP)], rbuf.at[pl.ds(0, _CP)])
            iota = lax.iota(jnp.int32, 16)
            zero16 = lax.shift_right_logical(iota, 4)
            for half in range(2):

                def gstep(g, carry2):
                    r = iota + 16 * g + half * _HR
                    q16 = lax.shift_right_logical(r, 2)
                    c16 = lax.shift_left(r & 3, 5)
                    rb = lax.shift_right_logical(r * 83887, 22)
                    s16 = r + 6 * rb - half * _HB * _HP
                    for c in range(dim):
                        v = plsc.load_gather(rbuf, [q16, c16 + c])
                        plsc.store_scatter(cbuf, [s16, zero16 + c], v)
                    return carry2

                lax.fori_loop(0, _HR // 16, gstep, None)
                for k in range(_HB):
                    pltpu.sync_copy(
                        cbuf.at[pl.ds(k * _HP, h_sz)],
                        out_hbm.at[b0 + ch * _CB + half * _HB + k],
                    )
            return carry

        lax.fori_loop(0, nchunk, step, None)

    return format_kernel


def kernel(input, emb_matrix):
    b_sz, h_sz = input.shape
    batch = b_sz * h_sz
    vocab, dim = emb_matrix.shape
    idx = input.reshape(batch // _GL, _GL).astype(jnp.int32)
    flat = _make_gather(vocab, batch, dim)(emb_matrix, idx)
    g128 = jax.lax.optimization_barrier(flat.reshape(batch // 4, dim * 4))
    return g128.reshape(b_sz, h_sz, dim)


# speedup vs baseline: 1.6280x; 1.0001x over previous
import functools

import jax
import jax.numpy as jnp
from jax import lax
from jax.experimental import pallas as pl
from jax.experimental.pallas import tpu as pltpu
from jax.experimental.pallas import tpu_sc as plsc

_NC = 2
_NS = 16
_NW = _NC * _NS

_GL = 128
_KG = 10
_BLK = _GL * _KG


def _make_gather(vocab: int, batch: int, dim: int):
    assert batch % (_NW * 2 * _BLK) == 0
    rows_per_w = batch // _NW
    irows_per_w = rows_per_w // _GL
    npair = rows_per_w // (2 * _BLK)

    mesh = plsc.VectorSubcoreMesh(core_axis_name="c", subcore_axis_name="s")

    @functools.partial(
        pl.kernel,
        out_type=jax.ShapeDtypeStruct((batch, dim), jnp.float32),
        mesh=mesh,
        scratch_types=[
            pltpu.VMEM((irows_per_w, _GL), jnp.int32),
            pltpu.VMEM((_BLK, dim), jnp.float32),
            pltpu.VMEM((_BLK, dim), jnp.float32),
            pltpu.SemaphoreType.DMA,
            pltpu.SemaphoreType.DMA,
            pltpu.SemaphoreType.DMA,
            pltpu.SemaphoreType.DMA,
        ],
        compiler_params=pltpu.CompilerParams(use_tc_tiling_on_sc=False),
    )
    def gather_kernel(table_hbm, idx_hbm, out_hbm, idx_v, buf0, buf1,
                      gsem0, gsem1, wsem0, wsem1):
        wid = lax.axis_index("s") * _NC + lax.axis_index("c")
        row0 = wid * rows_per_w
        pltpu.sync_copy(idx_hbm.at[pl.ds(wid * irows_per_w, irows_per_w)], idx_v)

        def fire_gather(blk, buf, sem):
            for j in range(_KG):
                pltpu.async_copy(
                    table_hbm.at[idx_v.at[blk * _KG + j]],
                    buf.at[pl.ds(j * _GL, _GL)],
                    sem,
                )

        def drain_gather(buf, sem):
            pltpu.make_async_copy(table_hbm.at[pl.ds(0, _BLK)], buf, sem).wait()

        def fire_write(blk, buf, sem):
            pltpu.async_copy(buf, out_hbm.at[pl.ds(row0 + blk * _BLK, _BLK)], sem)

        def drain_write(buf, sem):
            pltpu.make_async_copy(buf, out_hbm.at[pl.ds(0, _BLK)], sem).wait()

        fire_gather(0, buf0, gsem0)

        def step(p, carry):
            @pl.when(p >= 1)
            def _():
                drain_write(buf1, wsem1)
            fire_gather(2 * p + 1, buf1, gsem1)
            drain_gather(buf0, gsem0)
            fire_write(2 * p, buf0, wsem0)

            @pl.when(p <= npair - 2)
            def _():
                drain_write(buf0, wsem0)
                fire_gather(2 * p + 2, buf0, gsem0)
            drain_gather(buf1, gsem1)
            fire_write(2 * p + 1, buf1, wsem1)
            return carry

        lax.fori_loop(0, npair, step, None)
        drain_write(buf0, wsem0)
        drain_write(buf1, wsem1)

    return gather_kernel


def kernel(input, emb_matrix):
    b_sz, h_sz = input.shape
    batch = b_sz * h_sz
    vocab, dim = emb_matrix.shape
    idx = input.reshape(batch // _GL, _GL).astype(jnp.int32)
    flat = _make_gather(vocab, batch, dim)(emb_matrix, idx)
    g128 = jax.lax.optimization_barrier(flat.reshape(batch // 4, dim * 4))
    return g128.reshape(b_sz, h_sz, dim)
